# baseline (speedup 1.0000x reference)
import functools

import jax
import jax.numpy as jnp
import numpy as np
from jax import lax
from jax.experimental import pallas as pl
from jax.experimental.pallas import tpu as pltpu
from jax.experimental.pallas import tpu_sc as plsc

_INDEX_SHIFT = np.array(
    [[0, 0], [-1, 0], [1, 0], [0, 1], [-1, 1], [1, 1], [0, -1], [-1, -1], [1, -1]],
    dtype=np.int32,
)
_H, _W = 496, 432
_C = 128
_N = 20000


def _qkv_body(lf_ref, rf_ref, wq1, wk1, wv1, wq2, wk2, wv2,
              q1, k1, v1, q2, k2, v2):
    lf = lf_ref[...]
    rf = rf_ref[...]
    q1[...] = jnp.dot(lf, wq1[...], preferred_element_type=jnp.float32)
    k1[...] = jnp.dot(rf, wk1[...], preferred_element_type=jnp.float32)
    v1[...] = jnp.dot(rf, wv1[...], preferred_element_type=jnp.float32)
    q2[...] = jnp.dot(rf, wq2[...], preferred_element_type=jnp.float32)
    k2[...] = jnp.dot(lf, wk2[...], preferred_element_type=jnp.float32)
    v2[...] = jnp.dot(lf, wv2[...], preferred_element_type=jnp.float32)


def _qkv(lf, rf, Wq1, Wk1, Wv1, Wq2, Wk2, Wv2):
    n = lf.shape[0]
    blk = 2000
    grid = (n // blk,)
    row_spec = pl.BlockSpec((blk, _C), lambda i: (i, 0))
    w_spec = pl.BlockSpec((_C, _C), lambda i: (0, 0))
    out_sd = jax.ShapeDtypeStruct((n, _C), jnp.float32)
    return pl.pallas_call(
        _qkv_body,
        grid=grid,
        in_specs=[row_spec, row_spec] + [w_spec] * 6,
        out_specs=[row_spec] * 6,
        out_shape=[out_sd] * 6,
    )(lf, rf, Wq1, Wk1, Wv1, Wq2, Wk2, Wv2)


_NC, _NS, _L = 2, 16, 16
_NWORK = _NC * _NS
_NP = 20480
_PPW = _NP // _NWORK
_HWP = 215040
_CPW = _HWP // _NWORK
_CPG = _HWP // 16
_DUMMY = _NP - _L


def _mesh():
    return plsc.VectorSubcoreMesh(core_axis_name="c", subcore_axis_name="s")


_SC_PARAMS = pltpu.CompilerParams(needs_layout_passes=False)


def _iota():
    return lax.iota(jnp.int32, 16)


def _shift_up(x):
    idx = jnp.minimum(_iota() + 1, 15).reshape(16, 1)
    dn = lax.GatherDimensionNumbers(
        offset_dims=(), collapsed_slice_dims=(0,), start_index_map=(0,))
    return lax.gather(x, idx, dn, (1,), mode=lax.GatherScatterMode.PROMISE_IN_BOUNDS)


def _grid_build_body(lc_hbm, rc_hbm, gli_hbm, gra_hbm, cbuf, gbuf, sem):
    wid = lax.axis_index("s") * _NC + lax.axis_index("c")
    gslot = wid % 16
    is_li = wid < 16
    cell_lo = gslot * _CPG
    iota = _iota()
    big = jnp.int32(1 << 24)

    def init_step(v, _):
        gbuf[pl.ds(v * 16, 16)] = jnp.full((16,), -1, jnp.int32)
        return _
    lax.fori_loop(0, _CPG // 16, init_step, 0)

    def scan(coords_hbm):
        nstage = _NP // 256

        def stage_step(s, _):
            pltpu.sync_copy(coords_hbm.at[pl.ds(s * 512, 512)], cbuf)

            def vec_step(v, _):
                base = v * 16
                ii = (iota + base) * 2
                c0 = plsc.load_gather(cbuf, [ii])
                c1 = plsc.load_gather(cbuf, [ii + 1])
                lin = c0 * _W + c1
                rel = lin - cell_lo
                m_in = (rel >= 0) & (rel < _CPG)
                n_vec = s * 256 + base + iota
                m_in = m_in & (n_vec < _N)
                key = jnp.where(m_in, rel * 16 + iota, big + iota)
                ks, vs = plsc.sort_key_val(key, n_vec)
                addr = ks >> 4
                nxt = _shift_up(addr)
                winner = ((addr != nxt) | (iota == 15)) & (ks < big)
                plsc.store_scatter(gbuf, [addr], vs, mask=winner)
                return _
            lax.fori_loop(0, 16, vec_step, 0)
            return _
        lax.fori_loop(0, nstage, stage_step, 0)

    @pl.when(is_li)
    def _():
        scan(lc_hbm)
        pltpu.sync_copy(gbuf, gli_hbm.at[pl.ds(cell_lo, _CPG)])

    @pl.when(jnp.logical_not(is_li))
    def _():
        scan(rc_hbm)
        pltpu.sync_copy(gbuf, gra_hbm.at[pl.ds(cell_lo, _CPG)])


def _build_grids(lc_pad, rc_pad):
    f = pl.kernel(
        _grid_build_body,
        out_type=[jax.ShapeDtypeStruct((_HWP,), jnp.int32),
                  jax.ShapeDtypeStruct((_HWP,), jnp.int32)],
        mesh=_mesh(),
        compiler_params=_SC_PARAMS,
        scratch_types=[
            pltpu.VMEM((512,), jnp.int32),
            pltpu.VMEM((_CPG,), jnp.int32),
            pltpu.SemaphoreType.DMA,
        ],
    )
    return f(lc_pad, rc_pad)


def _bcast_lane(v, i):
    dn = lax.GatherDimensionNumbers(
        offset_dims=(), collapsed_slice_dims=(0,), start_index_map=(0,))
    return lax.gather(v, jnp.full((16, 1), i, jnp.int32), dn, (1,),
                      mode=lax.GatherScatterMode.PROMISE_IN_BOUNDS)


def _unpack_bf16(w):
    lo = plsc.bitcast(w << 16, jnp.float32)
    hi = plsc.bitcast(w & jnp.int32(-65536), jnp.float32)
    return lo, hi


def _attn_body(lcf, rcf, gli, gra, q1, kv1, q2, kv2, pos,
               o1, o2, cbuf, posb, qb, linb, selraw, selb, mskb,
               kvb0, kvb1, kvb2, outb, sem):
    wid = lax.axis_index("s") * _NC + lax.axis_index("c")
    iota = _iota()
    scale = jnp.float32(1.0 / np.sqrt(128.0))
    pltpu.sync_copy(pos, posb)
    kvbs = (kvb0, kvb1, kvb2)

    for (cf, grid, q, kv, o) in ((lcf, gra, q1, kv1, o1),
                                 (rcf, gli, q2, kv2, o2)):
        pltpu.sync_copy(cf.at[pl.ds(wid * 1280, 1280)], cbuf)

        def blk_step(blk, _, cf=cf, grid=grid, q=q, kv=kv, o=o):
            p0 = wid * 640 + blk * 32
            pltpu.sync_copy(q.at[pl.ds(p0, 32)], qb)
            for g in range(2):
                ii = (iota + blk * 32 + g * 16) * 2
                c0 = plsc.load_gather(cbuf, [ii])
                c1 = plsc.load_gather(cbuf, [ii + 1])
                lin = c0 * _W + c1
                for i in range(9):
                    dy, dx = int(_INDEX_SHIFT[i, 0]), int(_INDEX_SHIFT[i, 1])
                    sl = lin + (dy * _W + dx)
                    m = iota >= 0
                    if dy < 0:
                        m = m & (c0 >= 1)
                    if dx < 0:
                        m = m & (c1 >= 1)
                    if dx > 0:
                        m = m & (c1 <= _W - 2)
                    j = i * 32 + g * 16
                    linb[j // 96, pl.ds(j % 96, 16)] = jnp.where(m, sl, 0)
                    mskb[pl.ds(j, 16)] = m.astype(jnp.float32)
            descs = [pltpu.async_copy(grid.at[linb.at[j3]],
                                      selraw.at[pl.ds(j3 * 96, 96)], sem)
                     for j3 in range(3)]
            for d in descs:
                d.wait()
            for jv in range(18):
                sel = selraw[pl.ds(jv * 16, 16)]
                m = mskb[pl.ds(jv * 16, 16)]
                mf = jnp.where(sel >= 0, m, 0.0)
                safe = jnp.maximum(sel, 0)
                j = jv * 16
                selb[pl.ds(j, 16)] = safe
                mskb[pl.ds(j, 16)] = mf
            descs = [pltpu.async_copy(kv.at[selb.at[pl.ds(j3 * 96, 96)]],
                                      kvbs[j3], sem)
                     for j3 in range(3)]
            for d in descs:
                d.wait()
            for g in range(2):
                rowg = g * 16

                def kstep(c2, accs, rowg=rowg):
                    col = jnp.full((16,), c2, jnp.int32)
                    qv0 = plsc.load_gather(qb, [rowg + iota, 2 * col])
                    qv1 = plsc.load_gather(qb, [rowg + iota, 2 * col + 1])
                    out = []
                    for i in range(9):
                        j = i * 32 + rowg
                        kw = plsc.load_gather(kvbs[j // 96],
                                              [j % 96 + iota, col])
                        k0, k1 = _unpack_bf16(kw)
                        out.append(accs[i] + qv0 * k0 + qv1 * k1)
                    return tuple(out)

                accs = lax.fori_loop(
                    0, 64, kstep,
                    tuple(jnp.zeros((16,), jnp.float32) for _ in range(9)))
                msks = [mskb[pl.ds(i * 32 + rowg, 16)] for i in range(9)]
                logits = [accs[i] * (msks[i] * scale) for i in range(9)]
                mx = logits[0]
                for l in logits[1:]:
                    mx = jnp.maximum(mx, l)
                es = [jnp.exp(l - mx) for l in logits]
                s = es[0]
                for e in es[1:]:
                    s = s + e
                r = 1.0 / s
                wms = [es[i] * r * msks[i] for i in range(9)]
                pvalid = ((p0 + rowg + iota) < _N).astype(jnp.float32)

                def vstep(c2, carry, rowg=rowg, wms=wms, pvalid=pvalid):
                    col = jnp.full((16,), c2, jnp.int32)
                    pg0 = plsc.load_gather(posb, [iota, 2 * col])
                    pg1 = plsc.load_gather(posb, [iota, 2 * col + 1])
                    acc0 = jnp.zeros((16,), jnp.float32)
                    acc1 = jnp.zeros((16,), jnp.float32)
                    for i in range(9):
                        j = i * 32 + rowg
                        vw = plsc.load_gather(
                            kvbs[j // 96], [j % 96 + iota, 64 + col])
                        v0, v1 = _unpack_bf16(vw)
                        acc0 = acc0 + wms[i] * (v0 + _bcast_lane(pg0, i))
                        acc1 = acc1 + wms[i] * (v1 + _bcast_lane(pg1, i))
                    plsc.store_scatter(outb, [rowg + iota, 2 * col],
                                       acc0 * pvalid)
                    plsc.store_scatter(outb, [rowg + iota, 2 * col + 1],
                                       acc1 * pvalid)
                    return carry

                lax.fori_loop(0, 64, vstep, 0)
            pltpu.sync_copy(outb, o.at[pl.ds(p0, 32)])
            return _

        lax.fori_loop(0, 20, blk_step, 0)


def _attn(lc_flat, rc_flat, gli, gra, q1p, kv1, q2p, kv2, pos16):
    f = pl.kernel(
        _attn_body,
        out_type=[jax.ShapeDtypeStruct((_NP, _C), jnp.float32),
                  jax.ShapeDtypeStruct((_NP, _C), jnp.float32)],
        mesh=_mesh(),
        compiler_params=_SC_PARAMS,
        scratch_types=[
            pltpu.VMEM((1280,), jnp.int32),
            pltpu.VMEM((16, 128), jnp.float32),
            pltpu.VMEM((32, 128), jnp.float32),
            pltpu.VMEM((3, 96), jnp.int32),
            pltpu.VMEM((288,), jnp.int32),
            pltpu.VMEM((288,), jnp.int32),
            pltpu.VMEM((288,), jnp.float32),
            pltpu.VMEM((96, 128), jnp.int32),
            pltpu.VMEM((96, 128), jnp.int32),
            pltpu.VMEM((96, 128), jnp.int32),
            pltpu.VMEM((32, 128), jnp.float32),
            pltpu.SemaphoreType.DMA,
        ],
    )
    return f(lc_flat, rc_flat, gli, gra, q1p, kv1, q2p, kv2, pos16)


_NCH = _CPW // 96


def _canvas_body(gli, gra, out1, out2, zz, ct1, ct2, gb, widx, cellb, zbuf,
                 rows, sem):
    wid = lax.axis_index("s") * _NC + lax.axis_index("c")
    cell0 = wid * _CPW
    iota = _iota()
    pltpu.sync_copy(zz, zbuf)
    for (grid, out, ct) in ((gli, out1, ct1), (gra, out2, ct2)):
        pltpu.sync_copy(grid.at[pl.ds(cell0, _CPW)], gb)

        def cmp_step(v, cnt):
            g16 = gb[pl.ds(v * 16, 16)]
            m = g16 >= 0
            slots = cnt + plsc.cumsum(m.astype(jnp.int32)) - 1
            plsc.store_scatter(widx, [slots], g16, mask=m)
            plsc.store_scatter(cellb, [slots // 96, slots % 96],
                              cell0 + v * 16 + iota, mask=m)
            return cnt + plsc.all_reduce_population_count(m)

        cntv = lax.fori_loop(0, _CPW // 16, cmp_step,
                             jnp.zeros((16,), jnp.int32))
        for u in range(6):
            slots = cntv + u * 16 + iota
            plsc.store_scatter(widx, [slots],
                               jnp.full((16,), _DUMMY, jnp.int32))
            plsc.store_scatter(cellb, [slots // 96, slots % 96],
                               jnp.full((16,), _HWP, jnp.int32))
        cnt = jnp.max(cntv)

        def z_step(ch, carry, ct=ct):
            pltpu.sync_copy(zbuf, ct.at[pl.ds(cell0 + ch * 96, 96)])
            return carry

        lax.fori_loop(0, _NCH, z_step, 0)

        def g_step(ch, carry, out=out, ct=ct):
            pltpu.async_copy(out.at[widx.at[pl.ds(ch * 96, 96)]], rows,
                             sem).wait()
            pltpu.async_copy(rows, ct.at[cellb.at[ch]], sem).wait()
            return carry

        lax.fori_loop(0, (cnt + 95) // 96, g_step, 0)


def _canvas_sc(gli, gra, out1, out2):
    zz = jnp.zeros((96, _C), jnp.float32)
    f = pl.kernel(
        _canvas_body,
        out_type=[jax.ShapeDtypeStruct((_HWP + 8, _C), jnp.float32),
                  jax.ShapeDtypeStruct((_HWP + 8, _C), jnp.float32)],
        mesh=_mesh(),
        compiler_params=_SC_PARAMS,
        scratch_types=[
            pltpu.VMEM((_CPW,), jnp.int32),
            pltpu.VMEM((_CPW + 96,), jnp.int32),
            pltpu.VMEM((_NCH + 1, 96), jnp.int32),
            pltpu.VMEM((96, 128), jnp.float32),
            pltpu.VMEM((96, 128), jnp.float32),
            pltpu.SemaphoreType.DMA,
        ],
    )
    return f(gli, gra, out1, out2, zz)


def _transpose_body(t1, t2, o1, o2):
    o1[...] = t1[...].T
    o2[...] = t2[...].T


def _transpose(ct1, ct2):
    blk = 384
    nblk = (_H * _W) // blk
    f = pl.pallas_call(
        _transpose_body,
        grid=(nblk,),
        in_specs=[pl.BlockSpec((blk, _C), lambda i: (i, 0))] * 2,
        out_specs=[pl.BlockSpec((_C, blk), lambda i: (0, i))] * 2,
        out_shape=[jax.ShapeDtypeStruct((_C, _H * _W), jnp.float32)] * 2,
    )
    return f(ct1, ct2)


def _pack_kv(k, v):
    kv = jnp.concatenate([k.astype(jnp.bfloat16), v.astype(jnp.bfloat16)],
                         axis=1)
    return jax.lax.bitcast_convert_type(kv.reshape(-1, _C, 2), jnp.int32)


def kernel(li_bev_feats, li_bev_coors, ra_bev_feats, ra_bev_coors,
           pos_embedding, Wq1, Wk1, Wv1, Wq2, Wk2, Wv2):
    lf, lc = li_bev_feats[0], li_bev_coors[0]
    rf, rc = ra_bev_feats[0], ra_bev_coors[0]
    q1, k1, v1, q2, k2, v2 = _qkv(lf, rf, Wq1, Wk1, Wv1, Wq2, Wk2, Wv2)
    pad = ((0, _NP - _N), (0, 0))
    lc_flat = jnp.pad(lc, pad).reshape(-1)
    rc_flat = jnp.pad(rc, pad).reshape(-1)
    grid_li, grid_ra = _build_grids(lc_flat, rc_flat)
    q1p = jnp.pad(q1, pad)
    q2p = jnp.pad(q2, pad)
    kv1 = _pack_kv(k1, v1)
    kv2 = _pack_kv(k2, v2)
    pos16 = jnp.pad(pos_embedding, ((0, 7), (0, 0)))
    out1, out2 = _attn(lc_flat, rc_flat, grid_li, grid_ra,
                       q1p, kv1, q2p, kv2, pos16)
    ct1, ct2 = _canvas_sc(grid_li, grid_ra, out1, out2)
    c1, c2 = _transpose(ct1, ct2)
    return (c1.reshape(1, _C, _H, _W), c2.reshape(1, _C, _H, _W))


# speedup vs baseline: 1.0040x; 1.0040x over previous
import functools

import jax
import jax.numpy as jnp
import numpy as np
from jax import lax
from jax.experimental import pallas as pl
from jax.experimental.pallas import tpu as pltpu
from jax.experimental.pallas import tpu_sc as plsc

_INDEX_SHIFT = np.array(
    [[0, 0], [-1, 0], [1, 0], [0, 1], [-1, 1], [1, 1], [0, -1], [-1, -1], [1, -1]],
    dtype=np.int32,
)
_H, _W = 496, 432
_C = 128
_N = 20000


def _qkv_body(lf_ref, rf_ref, wq1, wk1, wv1, wq2, wk2, wv2,
              q1, k1, v1, q2, k2, v2):
    lf = lf_ref[...]
    rf = rf_ref[...]
    q1[...] = jnp.dot(lf, wq1[...], preferred_element_type=jnp.float32)
    k1[...] = jnp.dot(rf, wk1[...], preferred_element_type=jnp.float32)
    v1[...] = jnp.dot(rf, wv1[...], preferred_element_type=jnp.float32)
    q2[...] = jnp.dot(rf, wq2[...], preferred_element_type=jnp.float32)
    k2[...] = jnp.dot(lf, wk2[...], preferred_element_type=jnp.float32)
    v2[...] = jnp.dot(lf, wv2[...], preferred_element_type=jnp.float32)


def _qkv(lf, rf, Wq1, Wk1, Wv1, Wq2, Wk2, Wv2):
    n = lf.shape[0]
    blk = 2000
    grid = (n // blk,)
    row_spec = pl.BlockSpec((blk, _C), lambda i: (i, 0))
    w_spec = pl.BlockSpec((_C, _C), lambda i: (0, 0))
    out_sd = jax.ShapeDtypeStruct((n, _C), jnp.float32)
    return pl.pallas_call(
        _qkv_body,
        grid=grid,
        in_specs=[row_spec, row_spec] + [w_spec] * 6,
        out_specs=[row_spec] * 6,
        out_shape=[out_sd] * 6,
    )(lf, rf, Wq1, Wk1, Wv1, Wq2, Wk2, Wv2)


_NC, _NS, _L = 2, 16, 16
_NWORK = _NC * _NS
_NP = 20480
_PPW = _NP // _NWORK
_HWP = 215040
_CPW = _HWP // _NWORK
_CPG = _HWP // 16
_DUMMY = _NP - _L


def _mesh():
    return plsc.VectorSubcoreMesh(core_axis_name="c", subcore_axis_name="s")


_SC_PARAMS = pltpu.CompilerParams(needs_layout_passes=False)


def _iota():
    return lax.iota(jnp.int32, 16)


def _shift_up(x):
    idx = jnp.minimum(_iota() + 1, 15).reshape(16, 1)
    dn = lax.GatherDimensionNumbers(
        offset_dims=(), collapsed_slice_dims=(0,), start_index_map=(0,))
    return lax.gather(x, idx, dn, (1,), mode=lax.GatherScatterMode.PROMISE_IN_BOUNDS)


def _grid_build_body(lc_hbm, rc_hbm, gli_hbm, gra_hbm, cbuf, gbuf, sem):
    wid = lax.axis_index("s") * _NC + lax.axis_index("c")
    gslot = wid % 16
    is_li = wid < 16
    cell_lo = gslot * _CPG
    iota = _iota()
    big = jnp.int32(1 << 24)

    def init_step(v, _):
        gbuf[pl.ds(v * 16, 16)] = jnp.full((16,), -1, jnp.int32)
        return _
    lax.fori_loop(0, _CPG // 16, init_step, 0)

    def scan(coords_hbm):
        nstage = _NP // 256

        def stage_step(s, _):
            pltpu.sync_copy(coords_hbm.at[pl.ds(s * 512, 512)], cbuf)

            def vec_step(v, _):
                base = v * 16
                ii = (iota + base) * 2
                c0 = plsc.load_gather(cbuf, [ii])
                c1 = plsc.load_gather(cbuf, [ii + 1])
                lin = c0 * _W + c1
                rel = lin - cell_lo
                m_in = (rel >= 0) & (rel < _CPG)
                n_vec = s * 256 + base + iota
                m_in = m_in & (n_vec < _N)
                key = jnp.where(m_in, rel * 16 + iota, big + iota)
                ks, vs = plsc.sort_key_val(key, n_vec)
                addr = ks >> 4
                nxt = _shift_up(addr)
                winner = ((addr != nxt) | (iota == 15)) & (ks < big)
                plsc.store_scatter(gbuf, [addr], vs, mask=winner)
                return _
            lax.fori_loop(0, 16, vec_step, 0)
            return _
        lax.fori_loop(0, nstage, stage_step, 0)

    @pl.when(is_li)
    def _():
        scan(lc_hbm)
        pltpu.sync_copy(gbuf, gli_hbm.at[pl.ds(cell_lo, _CPG)])

    @pl.when(jnp.logical_not(is_li))
    def _():
        scan(rc_hbm)
        pltpu.sync_copy(gbuf, gra_hbm.at[pl.ds(cell_lo, _CPG)])


def _build_grids(lc_pad, rc_pad):
    f = pl.kernel(
        _grid_build_body,
        out_type=[jax.ShapeDtypeStruct((_HWP,), jnp.int32),
                  jax.ShapeDtypeStruct((_HWP,), jnp.int32)],
        mesh=_mesh(),
        compiler_params=_SC_PARAMS,
        scratch_types=[
            pltpu.VMEM((512,), jnp.int32),
            pltpu.VMEM((_CPG,), jnp.int32),
            pltpu.SemaphoreType.DMA,
        ],
    )
    return f(lc_pad, rc_pad)


def _bcast_lane(v, i):
    dn = lax.GatherDimensionNumbers(
        offset_dims=(), collapsed_slice_dims=(0,), start_index_map=(0,))
    return lax.gather(v, jnp.full((16, 1), i, jnp.int32), dn, (1,),
                      mode=lax.GatherScatterMode.PROMISE_IN_BOUNDS)


def _unpack_bf16(w):
    lo = plsc.bitcast(w << 16, jnp.float32)
    hi = plsc.bitcast(w & jnp.int32(-65536), jnp.float32)
    return lo, hi


def _attn_body(lcf, rcf, gli, gra, q1, kv1, q2, kv2, pos,
               o1, o2, cbuf, posb, qb, linb, selraw, selb, mskb,
               kvb0, kvb1, kvb2, kvb3, kvb4, kvb5, outb, sem):
    wid = lax.axis_index("s") * _NC + lax.axis_index("c")
    iota = _iota()
    scale = jnp.float32(1.0 / np.sqrt(128.0))
    pltpu.sync_copy(pos, posb)
    kvbs = (kvb0, kvb1, kvb2, kvb3, kvb4, kvb5)

    for (cf, grid, q, kv, o) in ((lcf, gra, q1, kv1, o1),
                                 (rcf, gli, q2, kv2, o2)):
        pltpu.sync_copy(cf.at[pl.ds(wid * 1280, 1280)], cbuf)

        def blk_step(blk, _, cf=cf, grid=grid, q=q, kv=kv, o=o):
            p0 = wid * 640 + blk * 64
            pltpu.sync_copy(q.at[pl.ds(p0, 64)], qb)
            for g in range(4):
                ii = (iota + blk * 64 + g * 16) * 2
                c0 = plsc.load_gather(cbuf, [ii])
                c1 = plsc.load_gather(cbuf, [ii + 1])
                lin = c0 * _W + c1
                for i in range(9):
                    dy, dx = int(_INDEX_SHIFT[i, 0]), int(_INDEX_SHIFT[i, 1])
                    sl = lin + (dy * _W + dx)
                    m = iota >= 0
                    if dy < 0:
                        m = m & (c0 >= 1)
                    if dx < 0:
                        m = m & (c1 >= 1)
                    if dx > 0:
                        m = m & (c1 <= _W - 2)
                    j = i * 64 + g * 16
                    linb[j // 96, pl.ds(j % 96, 16)] = jnp.where(m, sl, 0)
                    mskb[pl.ds(j, 16)] = m.astype(jnp.float32)
            descs = [pltpu.async_copy(grid.at[linb.at[j3]],
                                      selraw.at[pl.ds(j3 * 96, 96)], sem)
                     for j3 in range(6)]
            for d in descs:
                d.wait()
            for jv in range(36):
                sel = selraw[pl.ds(jv * 16, 16)]
                m = mskb[pl.ds(jv * 16, 16)]
                mf = jnp.where(sel >= 0, m, 0.0)
                safe = jnp.maximum(sel, 0)
                j = jv * 16
                selb[pl.ds(j, 16)] = safe
                mskb[pl.ds(j, 16)] = mf
            descs = [pltpu.async_copy(kv.at[selb.at[pl.ds(j3 * 96, 96)]],
                                      kvbs[j3], sem)
                     for j3 in range(6)]
            for d in descs:
                d.wait()
            for g in range(4):
                rowg = g * 16

                def kstep(c2, accs, rowg=rowg):
                    col = jnp.full((16,), c2, jnp.int32)
                    qv0 = plsc.load_gather(qb, [rowg + iota, 2 * col])
                    qv1 = plsc.load_gather(qb, [rowg + iota, 2 * col + 1])
                    out = []
                    for i in range(9):
                        j = i * 64 + rowg
                        kw = plsc.load_gather(kvbs[j // 96],
                                              [j % 96 + iota, col])
                        k0, k1 = _unpack_bf16(kw)
                        out.append(accs[i] + qv0 * k0 + qv1 * k1)
                    return tuple(out)

                accs = lax.fori_loop(
                    0, 64, kstep,
                    tuple(jnp.zeros((16,), jnp.float32) for _ in range(9)))
                msks = [mskb[pl.ds(i * 64 + rowg, 16)] for i in range(9)]
                logits = [accs[i] * (msks[i] * scale) for i in range(9)]
                mx = logits[0]
                for l in logits[1:]:
                    mx = jnp.maximum(mx, l)
                es = [jnp.exp(l - mx) for l in logits]
                s = es[0]
                for e in es[1:]:
                    s = s + e
                r = 1.0 / s
                wms = [es[i] * r * msks[i] for i in range(9)]
                pvalid = ((p0 + rowg + iota) < _N).astype(jnp.float32)

                def vstep(c2, carry, rowg=rowg, wms=wms, pvalid=pvalid):
                    col = jnp.full((16,), c2, jnp.int32)
                    pg0 = plsc.load_gather(posb, [iota, 2 * col])
                    pg1 = plsc.load_gather(posb, [iota, 2 * col + 1])
                    acc0 = jnp.zeros((16,), jnp.float32)
                    acc1 = jnp.zeros((16,), jnp.float32)
                    for i in range(9):
                        j = i * 64 + rowg
                        vw = plsc.load_gather(
                            kvbs[j // 96], [j % 96 + iota, 64 + col])
                        v0, v1 = _unpack_bf16(vw)
                        acc0 = acc0 + wms[i] * (v0 + _bcast_lane(pg0, i))
                        acc1 = acc1 + wms[i] * (v1 + _bcast_lane(pg1, i))
                    plsc.store_scatter(outb, [rowg + iota, 2 * col],
                                       acc0 * pvalid)
                    plsc.store_scatter(outb, [rowg + iota, 2 * col + 1],
                                       acc1 * pvalid)
                    return carry

                lax.fori_loop(0, 64, vstep, 0)
            pltpu.sync_copy(outb, o.at[pl.ds(p0, 64)])
            return _

        lax.fori_loop(0, 10, blk_step, 0)


def _attn(lc_flat, rc_flat, gli, gra, q1p, kv1, q2p, kv2, pos16):
    f = pl.kernel(
        _attn_body,
        out_type=[jax.ShapeDtypeStruct((_NP, _C), jnp.float32),
                  jax.ShapeDtypeStruct((_NP, _C), jnp.float32)],
        mesh=_mesh(),
        compiler_params=_SC_PARAMS,
        scratch_types=[
            pltpu.VMEM((1280,), jnp.int32),
            pltpu.VMEM((16, 128), jnp.float32),
            pltpu.VMEM((64, 128), jnp.float32),
            pltpu.VMEM((6, 96), jnp.int32),
            pltpu.VMEM((576,), jnp.int32),
            pltpu.VMEM((576,), jnp.int32),
            pltpu.VMEM((576,), jnp.float32),
            pltpu.VMEM((96, 128), jnp.int32),
            pltpu.VMEM((96, 128), jnp.int32),
            pltpu.VMEM((96, 128), jnp.int32),
            pltpu.VMEM((96, 128), jnp.int32),
            pltpu.VMEM((96, 128), jnp.int32),
            pltpu.VMEM((96, 128), jnp.int32),
            pltpu.VMEM((64, 128), jnp.float32),
            pltpu.SemaphoreType.DMA,
        ],
    )
    return f(lc_flat, rc_flat, gli, gra, q1p, kv1, q2p, kv2, pos16)


_NCH = _CPW // 96


def _canvas_body(gli, gra, out1, out2, zz, ct1, ct2, gb, widx, cellb, zbuf,
                 rows, sem):
    wid = lax.axis_index("s") * _NC + lax.axis_index("c")
    cell0 = wid * _CPW
    iota = _iota()
    pltpu.sync_copy(zz, zbuf)
    for (grid, out, ct) in ((gli, out1, ct1), (gra, out2, ct2)):
        pltpu.sync_copy(grid.at[pl.ds(cell0, _CPW)], gb)

        def cmp_step(v, cnt):
            g16 = gb[pl.ds(v * 16, 16)]
            m = g16 >= 0
            slots = cnt + plsc.cumsum(m.astype(jnp.int32)) - 1
            plsc.store_scatter(widx, [slots], g16, mask=m)
            plsc.store_scatter(cellb, [slots // 96, slots % 96],
                              cell0 + v * 16 + iota, mask=m)
            return cnt + plsc.all_reduce_population_count(m)

        cntv = lax.fori_loop(0, _CPW // 16, cmp_step,
                             jnp.zeros((16,), jnp.int32))
        for u in range(6):
            slots = cntv + u * 16 + iota
            plsc.store_scatter(widx, [slots],
                               jnp.full((16,), _DUMMY, jnp.int32))
            plsc.store_scatter(cellb, [slots // 96, slots % 96],
                               jnp.full((16,), _HWP, jnp.int32))
        cnt = jnp.max(cntv)

        def z_step(ch, carry, ct=ct):
            pltpu.sync_copy(zbuf, ct.at[pl.ds(cell0 + ch * 96, 96)])
            return carry

        lax.fori_loop(0, _NCH, z_step, 0)

        def g_step(ch, carry, out=out, ct=ct):
            pltpu.async_copy(out.at[widx.at[pl.ds(ch * 96, 96)]], rows,
                             sem).wait()
            pltpu.async_copy(rows, ct.at[cellb.at[ch]], sem).wait()
            return carry

        lax.fori_loop(0, (cnt + 95) // 96, g_step, 0)


def _canvas_sc(gli, gra, out1, out2):
    zz = jnp.zeros((96, _C), jnp.float32)
    f = pl.kernel(
        _canvas_body,
        out_type=[jax.ShapeDtypeStruct((_HWP + 8, _C), jnp.float32),
                  jax.ShapeDtypeStruct((_HWP + 8, _C), jnp.float32)],
        mesh=_mesh(),
        compiler_params=_SC_PARAMS,
        scratch_types=[
            pltpu.VMEM((_CPW,), jnp.int32),
            pltpu.VMEM((_CPW + 96,), jnp.int32),
            pltpu.VMEM((_NCH + 1, 96), jnp.int32),
            pltpu.VMEM((96, 128), jnp.float32),
            pltpu.VMEM((96, 128), jnp.float32),
            pltpu.SemaphoreType.DMA,
        ],
    )
    return f(gli, gra, out1, out2, zz)


def _transpose_body(t1, t2, o1, o2):
    o1[...] = t1[...].T
    o2[...] = t2[...].T


def _transpose(ct1, ct2):
    blk = 384
    nblk = (_H * _W) // blk
    f = pl.pallas_call(
        _transpose_body,
        grid=(nblk,),
        in_specs=[pl.BlockSpec((blk, _C), lambda i: (i, 0))] * 2,
        out_specs=[pl.BlockSpec((_C, blk), lambda i: (0, i))] * 2,
        out_shape=[jax.ShapeDtypeStruct((_C, _H * _W), jnp.float32)] * 2,
    )
    return f(ct1, ct2)


def _pack_kv(k, v):
    kv = jnp.concatenate([k.astype(jnp.bfloat16), v.astype(jnp.bfloat16)],
                         axis=1)
    return jax.lax.bitcast_convert_type(kv.reshape(-1, _C, 2), jnp.int32)


def kernel(li_bev_feats, li_bev_coors, ra_bev_feats, ra_bev_coors,
           pos_embedding, Wq1, Wk1, Wv1, Wq2, Wk2, Wv2):
    lf, lc = li_bev_feats[0], li_bev_coors[0]
    rf, rc = ra_bev_feats[0], ra_bev_coors[0]
    q1, k1, v1, q2, k2, v2 = _qkv(lf, rf, Wq1, Wk1, Wv1, Wq2, Wk2, Wv2)
    pad = ((0, _NP - _N), (0, 0))
    lc_flat = jnp.pad(lc, pad).reshape(-1)
    rc_flat = jnp.pad(rc, pad).reshape(-1)
    grid_li, grid_ra = _build_grids(lc_flat, rc_flat)
    q1p = jnp.pad(q1, pad)
    q2p = jnp.pad(q2, pad)
    kv1 = _pack_kv(k1, v1)
    kv2 = _pack_kv(k2, v2)
    pos16 = jnp.pad(pos_embedding, ((0, 7), (0, 0)))
    out1, out2 = _attn(lc_flat, rc_flat, grid_li, grid_ra,
                       q1p, kv1, q2p, kv2, pos16)
    ct1, ct2 = _canvas_sc(grid_li, grid_ra, out1, out2)
    c1, c2 = _transpose(ct1, ct2)
    return (c1.reshape(1, _C, _H, _W), c2.reshape(1, _C, _H, _W))
